# Initial kernel scaffold; baseline (speedup 1.0000x reference)
import jax
import jax.numpy as jnp
from jax.experimental import pallas as pl
from jax.experimental.pallas import tpu as pltpu

_D_MODEL = 1024
_D_INNER = 2048
_D_STATE = 64
_DT_RANK = 64
_D_CONV = 4
_LC = 256
_BLK = 8
_LOG2E = 1.4426950408889634


def _rep_sub(row):
    return jnp.tile(jnp.broadcast_to(row, (8, row.shape[1])), (8, 1))


def _rep_lane(col):
    return jnp.tile(jnp.broadcast_to(col, (64, 128)), (1, 16))


def _mamba_layer_kernel(x_ref, w_in_ref, cw_ref, cb_ref, xp_ref, dtw_ref,
                        dtb_ref, alog_ref, dskip_ref, w_out_ref, out_ref,
                        upre_ref, z_ref, u_ref, dt_ref, xdbl_ref, h_ref,
                        ys_ref):
    c = pl.program_id(1)

    @pl.when(c == 0)
    def _init():
        h_ref[...] = jnp.zeros_like(h_ref)
        upre_ref[0:8, :] = jnp.zeros((8, _D_INNER), jnp.float32)

    @pl.when(c > 0)
    def _carry_tail():
        upre_ref[0:8, :] = upre_ref[_LC:_LC + 8, :]

    x_bf = x_ref[0].astype(jnp.bfloat16)
    upre_ref[8:8 + _LC, :] = jnp.dot(
        x_bf, w_in_ref[:, :_D_INNER], preferred_element_type=jnp.float32)
    z_ref[...] = jnp.dot(
        x_bf, w_in_ref[:, _D_INNER:], preferred_element_type=jnp.float32)

    uc = (upre_ref[5:5 + _LC, :] * cw_ref[0:1, :]
          + upre_ref[6:6 + _LC, :] * cw_ref[1:2, :]
          + upre_ref[7:7 + _LC, :] * cw_ref[2:3, :]
          + upre_ref[8:8 + _LC, :] * cw_ref[3:4, :]) + cb_ref[...]
    u = uc * jax.nn.sigmoid(uc)
    u_ref[...] = u
    u_bf = u.astype(jnp.bfloat16)

    xdbl = jnp.dot(u_bf, xp_ref[...], preferred_element_type=jnp.float32)
    xdbl_ref[...] = xdbl
    dt_low_bf = xdbl[:, :_DT_RANK].astype(jnp.bfloat16)
    dt_pre = jnp.dot(dt_low_bf, dtw_ref[...],
                     preferred_element_type=jnp.float32) + dtb_ref[...]
    dt_ref[...] = jax.nn.softplus(dt_pre)

    a2 = -jnp.exp(alog_ref[...]) * _LOG2E

    def blk(j, carry):
        base = pl.multiple_of(j * _BLK, _BLK)
        dt8 = dt_ref[pl.ds(base, _BLK), :]
        u8 = u_ref[pl.ds(base, _BLK), :]
        w8 = dt8 * u8
        bc8 = xdbl_ref[pl.ds(base, _BLK), _DT_RANK:_DT_RANK + 2 * _D_STATE]
        t8 = bc8.T
        h = h_ref[...]
        ys = []
        for i in range(_BLK):
            dtf = _rep_sub(dt8[i:i + 1, :])
            da = jnp.exp2(dtf * a2)
            wf = _rep_sub(w8[i:i + 1, :])
            bfull = _rep_lane(t8[0:_D_STATE, i:i + 1])
            cfull = _rep_lane(t8[_D_STATE:2 * _D_STATE, i:i + 1])
            h = h * da + bfull * wf
            ys.append(jnp.sum(h * cfull, axis=0, keepdims=True))
        h_ref[...] = h
        ys_ref[pl.ds(base, _BLK), :] = jnp.concatenate(ys, axis=0)
        return carry

    jax.lax.fori_loop(0, _LC // _BLK, blk, 0)

    y = ys_ref[...] + u_ref[...] * dskip_ref[...]
    z = z_ref[...]
    y = y * (z * jax.nn.sigmoid(z))
    out_ref[0] = jnp.dot(y.astype(jnp.bfloat16), w_out_ref[...],
                         preferred_element_type=jnp.float32)


def _mamba_layer(x, in_w, cw, cb, xp_w, dtw, dtb, a_log, dskip, out_w):
    batch, seqlen, _ = x.shape
    nc = seqlen // _LC
    w_in_t = in_w.T.astype(jnp.bfloat16)
    cw_t = cw.T
    cb2 = cb.reshape(1, _D_INNER)
    xp_t = xp_w.T.astype(jnp.bfloat16)
    dtw_t = dtw.T.astype(jnp.bfloat16)
    dtb2 = dtb.reshape(1, _D_INNER)
    alog_t = a_log.T
    dskip2 = dskip.reshape(1, _D_INNER)
    w_out_t = out_w.T.astype(jnp.bfloat16)

    full = lambda shape: pl.BlockSpec(shape, lambda b, c: (0,) * len(shape))
    return pl.pallas_call(
        _mamba_layer_kernel,
        out_shape=jax.ShapeDtypeStruct((batch, seqlen, _D_MODEL), jnp.float32),
        grid=(batch, nc),
        in_specs=[
            pl.BlockSpec((1, _LC, _D_MODEL), lambda b, c: (b, c, 0)),
            full((_D_MODEL, 2 * _D_INNER)),
            full((_D_CONV, _D_INNER)),
            full((1, _D_INNER)),
            full((_D_INNER, _DT_RANK + 2 * _D_STATE)),
            full((_DT_RANK, _D_INNER)),
            full((1, _D_INNER)),
            full((_D_STATE, _D_INNER)),
            full((1, _D_INNER)),
            full((_D_INNER, _D_MODEL)),
        ],
        out_specs=pl.BlockSpec((1, _LC, _D_MODEL), lambda b, c: (b, c, 0)),
        scratch_shapes=[
            pltpu.VMEM((_LC + 8, _D_INNER), jnp.float32),
            pltpu.VMEM((_LC, _D_INNER), jnp.float32),
            pltpu.VMEM((_LC, _D_INNER), jnp.float32),
            pltpu.VMEM((_LC, _D_INNER), jnp.float32),
            pltpu.VMEM((_LC, _DT_RANK + 2 * _D_STATE), jnp.float32),
            pltpu.VMEM((_D_STATE, _D_INNER), jnp.float32),
            pltpu.VMEM((_LC, _D_INNER), jnp.float32),
        ],
        compiler_params=pltpu.CompilerParams(
            dimension_semantics=("core_parallel", "arbitrary"),
            vmem_limit_bytes=56 * 1024 * 1024,
        ),
        name="mamba_layer",
    )(x, w_in_t, cw_t, cb2, xp_t, dtw_t, dtb2, alog_t, dskip2, w_out_t)


def kernel(x, in_proj_w, conv_w, conv_b, x_proj_w, dt_proj_w, dt_proj_b,
           A_log, D_skip, out_proj_w):
    out = x
    for i in range(in_proj_w.shape[0]):
        out = _mamba_layer(out, in_proj_w[i], conv_w[i], conv_b[i],
                           x_proj_w[i], dt_proj_w[i], dt_proj_b[i],
                           A_log[i], D_skip[i], out_proj_w[i])
    return out


# speedup vs baseline: 12.0317x; 12.0317x over previous
import jax
import jax.numpy as jnp
from jax.experimental import pallas as pl
from jax.experimental.pallas import tpu as pltpu

_D_MODEL = 1024
_D_INNER = 2048
_D_STATE = 64
_DT_RANK = 64
_D_CONV = 4
_LC = 256
_BLK = 8
_LOG2E = 1.4426950408889634


def _rep_sub(row):
    return jnp.tile(jnp.broadcast_to(row, (8, row.shape[1])), (8, 1))


def _rep_lane(col):
    return jnp.tile(jnp.broadcast_to(col, (64, 128)), (1, 16))


def _mamba_layer_kernel(x_ref, w_in_ref, cw_ref, cb_ref, xp_ref, dtw_ref,
                        dtb_ref, alog_ref, dskip_ref, w_out_ref, out_ref,
                        upre_ref, z_ref, u_ref, dt_ref, xdbl_ref, h_ref,
                        ys_ref):
    c = pl.program_id(1)

    @pl.when(c == 0)
    def _init():
        h_ref[...] = jnp.zeros_like(h_ref)
        upre_ref[0:8, :] = jnp.zeros((8, _D_INNER), jnp.float32)

    @pl.when(c > 0)
    def _carry_tail():
        upre_ref[0:8, :] = upre_ref[_LC:_LC + 8, :]

    x_bf = x_ref[0].astype(jnp.bfloat16)
    upre_ref[8:8 + _LC, :] = jnp.dot(
        x_bf, w_in_ref[:, :_D_INNER], preferred_element_type=jnp.float32)
    z_ref[...] = jnp.dot(
        x_bf, w_in_ref[:, _D_INNER:], preferred_element_type=jnp.float32)

    uc = (upre_ref[5:5 + _LC, :] * cw_ref[0:1, :]
          + upre_ref[6:6 + _LC, :] * cw_ref[1:2, :]
          + upre_ref[7:7 + _LC, :] * cw_ref[2:3, :]
          + upre_ref[8:8 + _LC, :] * cw_ref[3:4, :]) + cb_ref[...]
    u = uc * jax.nn.sigmoid(uc)
    u_ref[...] = u
    u_bf = u.astype(jnp.bfloat16)

    xdbl = jnp.dot(u_bf, xp_ref[...], preferred_element_type=jnp.float32)
    xdbl_ref[...] = xdbl
    dt_low_bf = xdbl[:, :_DT_RANK].astype(jnp.bfloat16)
    dt_pre = jnp.dot(dt_low_bf, dtw_ref[...],
                     preferred_element_type=jnp.float32) + dtb_ref[...]
    dt_ref[...] = jax.nn.softplus(dt_pre)

    a2 = -jnp.exp(alog_ref[...]) * _LOG2E

    def blk(j, carry):
        base = pl.multiple_of(j * _BLK, _BLK)
        dt8 = dt_ref[pl.ds(base, _BLK), :]
        u8 = u_ref[pl.ds(base, _BLK), :]
        w8 = dt8 * u8
        bc8 = xdbl_ref[pl.ds(base, _BLK), _DT_RANK:_DT_RANK + 2 * _D_STATE]
        t8 = bc8.T
        h = h_ref[...]
        ys = []
        for i in range(_BLK):
            dtf = _rep_sub(dt8[i:i + 1, :])
            da = jnp.exp2(dtf * a2)
            wf = _rep_sub(w8[i:i + 1, :])
            bfull = _rep_lane(t8[0:_D_STATE, i:i + 1])
            cfull = _rep_lane(t8[_D_STATE:2 * _D_STATE, i:i + 1])
            h = h * da + bfull * wf
            ys.append(jnp.sum(h * cfull, axis=0, keepdims=True))
        h_ref[...] = h
        ys_ref[pl.ds(base, _BLK), :] = jnp.concatenate(ys, axis=0)
        return carry

    jax.lax.fori_loop(0, _LC // _BLK, blk, 0)

    y = ys_ref[...] + u_ref[...] * dskip_ref[...]
    z = z_ref[...]
    y = y * (z * jax.nn.sigmoid(z))
    out_ref[0] = jnp.dot(y.astype(jnp.bfloat16), w_out_ref[...],
                         preferred_element_type=jnp.float32)


def _mamba_layer(x, in_w, cw, cb, xp_w, dtw, dtb, a_log, dskip, out_w):
    batch, seqlen, _ = x.shape
    nc = seqlen // _LC
    w_in_t = in_w.T.astype(jnp.bfloat16)
    cw_t = cw.T
    cb2 = cb.reshape(1, _D_INNER)
    xp_t = xp_w.T.astype(jnp.bfloat16)
    dtw_t = dtw.T.astype(jnp.bfloat16)
    dtb2 = dtb.reshape(1, _D_INNER)
    alog_t = a_log.T
    dskip2 = dskip.reshape(1, _D_INNER)
    w_out_t = out_w.T.astype(jnp.bfloat16)

    full = lambda shape: pl.BlockSpec(shape, lambda b, c: (0,) * len(shape))
    return pl.pallas_call(
        _mamba_layer_kernel,
        out_shape=jax.ShapeDtypeStruct((batch, seqlen, _D_MODEL), jnp.float32),
        grid=(batch, nc),
        in_specs=[
            pl.BlockSpec((1, _LC, _D_MODEL), lambda b, c: (b, c, 0)),
            full((_D_MODEL, 2 * _D_INNER)),
            full((_D_CONV, _D_INNER)),
            full((1, _D_INNER)),
            full((_D_INNER, _DT_RANK + 2 * _D_STATE)),
            full((_DT_RANK, _D_INNER)),
            full((1, _D_INNER)),
            full((_D_STATE, _D_INNER)),
            full((1, _D_INNER)),
            full((_D_INNER, _D_MODEL)),
        ],
        out_specs=pl.BlockSpec((1, _LC, _D_MODEL), lambda b, c: (b, c, 0)),
        scratch_shapes=[
            pltpu.VMEM((_LC + 8, _D_INNER), jnp.float32),
            pltpu.VMEM((_LC, _D_INNER), jnp.float32),
            pltpu.VMEM((_LC, _D_INNER), jnp.float32),
            pltpu.VMEM((_LC, _D_INNER), jnp.float32),
            pltpu.VMEM((_LC, _DT_RANK + 2 * _D_STATE), jnp.float32),
            pltpu.VMEM((_D_STATE, _D_INNER), jnp.float32),
            pltpu.VMEM((_LC, _D_INNER), jnp.float32),
        ],
        compiler_params=pltpu.CompilerParams(
            dimension_semantics=("arbitrary", "arbitrary"),
            vmem_limit_bytes=56 * 1024 * 1024,
        ),
        name="mamba_layer",
    )(x, w_in_t, cw_t, cb2, xp_t, dtw_t, dtb2, alog_t, dskip2, w_out_t)


def kernel(x, in_proj_w, conv_w, conv_b, x_proj_w, dt_proj_w, dt_proj_b,
           A_log, D_skip, out_proj_w):
    out = x
    for i in range(in_proj_w.shape[0]):
        out = _mamba_layer(out, in_proj_w[i], conv_w[i], conv_b[i],
                           x_proj_w[i], dt_proj_w[i], dt_proj_b[i],
                           A_log[i], D_skip[i], out_proj_w[i])
    return out


# speedup vs baseline: 12.9284x; 1.0745x over previous
import jax
import jax.numpy as jnp
from jax.experimental import pallas as pl
from jax.experimental.pallas import tpu as pltpu

_D_MODEL = 1024
_D_INNER = 2048
_D_STATE = 64
_DT_RANK = 64
_D_CONV = 4
_NB = 2
_LC = 256
_BLK = 8
_LOG2E = 1.4426950408889634


def _rep_sub(row):
    return jnp.tile(jnp.broadcast_to(row, (8, row.shape[1])), (8, 1))


def _rep_lane(col):
    return jnp.tile(jnp.broadcast_to(col, (64, 128)), (1, 16))


def _mamba_layer_kernel(x_ref, w_in_ref, cw_ref, cb_ref, xp_ref, dtw_ref,
                        dtb_ref, alog_ref, dskip_ref, w_out_ref, out_ref,
                        upre_ref, z_ref, u_ref, dt_ref, xdbl_ref, h_ref,
                        ys_ref):
    c = pl.program_id(0)

    @pl.when(c == 0)
    def _init():
        h_ref[...] = jnp.zeros_like(h_ref)
        for b in range(_NB):
            upre_ref[b, 0:8, :] = jnp.zeros((8, _D_INNER), jnp.float32)

    @pl.when(c > 0)
    def _carry_tail():
        for b in range(_NB):
            upre_ref[b, 0:8, :] = upre_ref[b, _LC:_LC + 8, :]

    for b in range(_NB):
        x_bf = x_ref[b].astype(jnp.bfloat16)
        upre_ref[b, 8:8 + _LC, :] = jnp.dot(
            x_bf, w_in_ref[:, :_D_INNER], preferred_element_type=jnp.float32)
        z_ref[b] = jnp.dot(
            x_bf, w_in_ref[:, _D_INNER:], preferred_element_type=jnp.float32)

        uc = (upre_ref[b, 5:5 + _LC, :] * cw_ref[0:1, :]
              + upre_ref[b, 6:6 + _LC, :] * cw_ref[1:2, :]
              + upre_ref[b, 7:7 + _LC, :] * cw_ref[2:3, :]
              + upre_ref[b, 8:8 + _LC, :] * cw_ref[3:4, :]) + cb_ref[...]
        u = uc * jax.nn.sigmoid(uc)
        u_ref[b] = u
        u_bf = u.astype(jnp.bfloat16)

        xdbl = jnp.dot(u_bf, xp_ref[...], preferred_element_type=jnp.float32)
        xdbl_ref[b] = xdbl
        dt_low_bf = xdbl[:, :_DT_RANK].astype(jnp.bfloat16)
        dt_pre = jnp.dot(dt_low_bf, dtw_ref[...],
                         preferred_element_type=jnp.float32) + dtb_ref[...]
        dt_ref[b] = jax.nn.softplus(dt_pre)

    a2 = -jnp.exp(alog_ref[...]) * _LOG2E

    def blk(j, carry):
        base = pl.multiple_of(j * _BLK, _BLK)
        dt8 = [dt_ref[b, pl.ds(base, _BLK), :] for b in range(_NB)]
        w8 = [dt8[b] * u_ref[b, pl.ds(base, _BLK), :] for b in range(_NB)]
        t8 = [xdbl_ref[b, pl.ds(base, _BLK),
                       _DT_RANK:_DT_RANK + 2 * _D_STATE].T
              for b in range(_NB)]
        h = [h_ref[b] for b in range(_NB)]
        ys = [[] for _ in range(_NB)]
        for i in range(_BLK):
            for b in range(_NB):
                dtf = _rep_sub(dt8[b][i:i + 1, :])
                da = jnp.exp2(dtf * a2)
                wf = _rep_sub(w8[b][i:i + 1, :])
                bfull = _rep_lane(t8[b][0:_D_STATE, i:i + 1])
                cfull = _rep_lane(t8[b][_D_STATE:2 * _D_STATE, i:i + 1])
                h[b] = h[b] * da + bfull * wf
                ys[b].append(jnp.sum(h[b] * cfull, axis=0, keepdims=True))
        for b in range(_NB):
            h_ref[b] = h[b]
            ys_ref[b, pl.ds(base, _BLK), :] = jnp.concatenate(ys[b], axis=0)
        return carry

    jax.lax.fori_loop(0, _LC // _BLK, blk, 0)

    for b in range(_NB):
        y = ys_ref[b] + u_ref[b] * dskip_ref[...]
        z = z_ref[b]
        y = y * (z * jax.nn.sigmoid(z))
        out_ref[b] = jnp.dot(y.astype(jnp.bfloat16), w_out_ref[...],
                             preferred_element_type=jnp.float32)


def _mamba_layer(x, in_w, cw, cb, xp_w, dtw, dtb, a_log, dskip, out_w):
    batch, seqlen, _ = x.shape
    nc = seqlen // _LC
    w_in_t = in_w.T.astype(jnp.bfloat16)
    cw_t = cw.T
    cb2 = cb.reshape(1, _D_INNER)
    xp_t = xp_w.T.astype(jnp.bfloat16)
    dtw_t = dtw.T.astype(jnp.bfloat16)
    dtb2 = dtb.reshape(1, _D_INNER)
    alog_t = a_log.T
    dskip2 = dskip.reshape(1, _D_INNER)
    w_out_t = out_w.T.astype(jnp.bfloat16)

    full = lambda shape: pl.BlockSpec(shape, lambda c: (0,) * len(shape))
    return pl.pallas_call(
        _mamba_layer_kernel,
        out_shape=jax.ShapeDtypeStruct((batch, seqlen, _D_MODEL), jnp.float32),
        grid=(nc,),
        in_specs=[
            pl.BlockSpec((batch, _LC, _D_MODEL), lambda c: (0, c, 0)),
            full((_D_MODEL, 2 * _D_INNER)),
            full((_D_CONV, _D_INNER)),
            full((1, _D_INNER)),
            full((_D_INNER, _DT_RANK + 2 * _D_STATE)),
            full((_DT_RANK, _D_INNER)),
            full((1, _D_INNER)),
            full((_D_STATE, _D_INNER)),
            full((1, _D_INNER)),
            full((_D_INNER, _D_MODEL)),
        ],
        out_specs=pl.BlockSpec((batch, _LC, _D_MODEL), lambda c: (0, c, 0)),
        scratch_shapes=[
            pltpu.VMEM((_NB, _LC + 8, _D_INNER), jnp.float32),
            pltpu.VMEM((_NB, _LC, _D_INNER), jnp.float32),
            pltpu.VMEM((_NB, _LC, _D_INNER), jnp.float32),
            pltpu.VMEM((_NB, _LC, _D_INNER), jnp.float32),
            pltpu.VMEM((_NB, _LC, _DT_RANK + 2 * _D_STATE), jnp.float32),
            pltpu.VMEM((_NB, _D_STATE, _D_INNER), jnp.float32),
            pltpu.VMEM((_NB, _LC, _D_INNER), jnp.float32),
        ],
        compiler_params=pltpu.CompilerParams(
            dimension_semantics=("arbitrary",),
            vmem_limit_bytes=60 * 1024 * 1024,
        ),
        name="mamba_layer",
    )(x, w_in_t, cw_t, cb2, xp_t, dtw_t, dtb2, alog_t, dskip2, w_out_t)


def kernel(x, in_proj_w, conv_w, conv_b, x_proj_w, dt_proj_w, dt_proj_b,
           A_log, D_skip, out_proj_w):
    out = x
    for i in range(in_proj_w.shape[0]):
        out = _mamba_layer(out, in_proj_w[i], conv_w[i], conv_b[i],
                           x_proj_w[i], dt_proj_w[i], dt_proj_b[i],
                           A_log[i], D_skip[i], out_proj_w[i])
    return out
